# Initial kernel scaffold; baseline (speedup 1.0000x reference)
import jax
import jax.numpy as jnp
from jax.experimental import pallas as pl


def kernel(ploc, plabel, gtloc, gtlabel, dboxes):
    raise NotImplementedError("write your pallas kernel here")


# speedup vs baseline: 9.7296x; 9.7296x over previous
import jax
import jax.numpy as jnp
from jax import lax
from jax.experimental import pallas as pl
from jax.experimental.pallas import tpu as pltpu

B, D, NG, C = 64, 8732, 20, 81
THR = 0.5


def _per_batch_body(dbox_ref, gt_ref, glab_ref, ploc_ref, plabel_ref,
                    cn_ref, part_ref):
    db = dbox_ref[...]
    gt = gt_ref[0]
    glab = glab_ref[0]

    d_l, d_t, d_r, d_b = (db[:, 0:1], db[:, 1:2], db[:, 2:3], db[:, 3:4])
    g_l, g_t, g_r, g_b = (gt[0:1, :], gt[1:2, :], gt[2:3, :], gt[3:4, :])

    ilt_x = jnp.maximum(d_l, g_l)
    ilt_y = jnp.maximum(d_t, g_t)
    irb_x = jnp.minimum(d_r, g_r)
    irb_y = jnp.minimum(d_b, g_b)
    iw = jnp.clip(irb_x - ilt_x, 0.0)
    ih = jnp.clip(irb_y - ilt_y, 0.0)
    inter = iw * ih
    area_d = (d_r - d_l) * (d_b - d_t)
    area_g = (g_r - g_l) * (g_b - g_t)
    ious = inter / (area_d + area_g - inter + 1e-8)

    best = jnp.max(ious, axis=1, keepdims=True)
    lane = lax.broadcasted_iota(jnp.int32, (D, NG), 1)
    idx = jnp.min(jnp.where(ious == best, lane, NG), axis=1, keepdims=True)
    onehot = (lane == idx).astype(jnp.float32)

    mask = best > THR
    maskf = mask.astype(jnp.float32)
    pos_b = jnp.sum(maskf)

    mloc = lax.dot_general(onehot, gt, (((1,), (1,)), ((), ())),
                           preferred_element_type=jnp.float32)
    mlab = jnp.sum(onehot * glab, axis=1, keepdims=True)
    mlab = jnp.where(mask, mlab, 0.0)

    diff = ploc_ref[0] - mloc
    ad = jnp.abs(diff)
    sl1 = jnp.sum(jnp.where(ad < 1.0, 0.5 * diff * diff, ad - 0.5),
                  axis=1, keepdims=True)
    loss_l_b = jnp.sum(maskf * sl1)

    x = plabel_ref[0]
    m = jnp.max(x, axis=1, keepdims=True)
    lse = jnp.log(jnp.sum(jnp.exp(x - m), axis=1, keepdims=True)) + m
    cls_lane = lax.broadcasted_iota(jnp.int32, (D, C), 1)
    onehot_c = (cls_lane == mlab.astype(jnp.int32)).astype(jnp.float32)
    picked = jnp.sum(x * onehot_c, axis=1, keepdims=True)
    loss_c = lse - picked

    cn_ref[0] = jnp.where(mask, 0.0, loss_c)
    part_ref[0, 0, 0] = pos_b
    part_ref[0, 0, 1] = loss_l_b
    part_ref[0, 0, 2] = jnp.sum(maskf * loss_c)
    part_ref[0, 0, 3] = 0.0


def _mining_body(cn_ref, part_ref, out_ref):
    part = part_ref[...]
    pos = jnp.sum(part[:, :, 0:1])
    loss_l = jnp.sum(part[:, :, 1:2])
    pos_ce = jnp.sum(part[:, :, 2:3])

    kf = 3.0 * pos
    k_i = kf.astype(jnp.int32)

    def search_step(i, t):
        cand = t | (jnp.int32(1) << (jnp.int32(30) - i))
        bits = lax.bitcast_convert_type(cn_ref[...], jnp.int32)
        cnt = jnp.sum((bits >= cand).astype(jnp.int32))
        return jnp.where(cnt >= k_i, cand, t)

    t_bits = lax.fori_loop(0, 31, search_step, jnp.int32(0))

    cn = cn_ref[...]
    bits = lax.bitcast_convert_type(cn, jnp.int32)
    gt_m = bits > t_bits
    eq_m = bits == t_bits
    cnt_gt = jnp.sum(gt_m.astype(jnp.int32))
    s_gt = jnp.sum(jnp.where(gt_m, cn, 0.0))
    cnt_eq = jnp.sum(eq_m.astype(jnp.int32))
    s_eq = jnp.sum(jnp.where(eq_m, cn, 0.0))
    tval = s_eq / jnp.maximum(cnt_eq, 1).astype(jnp.float32)
    n_tie = jnp.clip(k_i - cnt_gt, 0, cnt_eq).astype(jnp.float32)
    neg_sum = s_gt + n_tie * tval
    neg_sum = jnp.where(k_i >= 1, neg_sum, 0.0)

    total = (loss_l + pos_ce + neg_sum) / jnp.maximum(pos, 1.0)
    out_ref[...] = jnp.full((1, 1), total, dtype=jnp.float32)


def kernel(ploc, plabel, gtloc, gtlabel, dboxes):
    gt_t = jnp.transpose(gtloc, (0, 2, 1))
    glab_f = gtlabel.astype(jnp.float32).reshape(B, 1, NG)

    cn, part = pl.pallas_call(
        _per_batch_body,
        grid=(B,),
        in_specs=[
            pl.BlockSpec((D, 4), lambda b: (0, 0)),
            pl.BlockSpec((1, 4, NG), lambda b: (b, 0, 0)),
            pl.BlockSpec((1, 1, NG), lambda b: (b, 0, 0)),
            pl.BlockSpec((1, D, 4), lambda b: (b, 0, 0)),
            pl.BlockSpec((1, D, C), lambda b: (b, 0, 0)),
        ],
        out_specs=[
            pl.BlockSpec((1, D, 1), lambda b: (b, 0, 0)),
            pl.BlockSpec((1, 1, 4), lambda b: (b, 0, 0),
                         memory_space=pltpu.SMEM),
        ],
        out_shape=[
            jax.ShapeDtypeStruct((B, D, 1), jnp.float32),
            jax.ShapeDtypeStruct((B, 1, 4), jnp.float32),
        ],
        compiler_params=pltpu.CompilerParams(
            dimension_semantics=("arbitrary",),
        ),
    )(dboxes, gt_t, glab_f, ploc, plabel)

    out = pl.pallas_call(
        _mining_body,
        out_shape=jax.ShapeDtypeStruct((1, 1), jnp.float32),
    )(cn.reshape(B, D), part)
    return out[0, 0]


# speedup vs baseline: 12.8760x; 1.3234x over previous
import jax
import jax.numpy as jnp
from jax import lax
from jax.experimental import pallas as pl
from jax.experimental.pallas import tpu as pltpu

B, D, NG, C = 64, 8732, 20, 81
DP = 8832
R = DP // 128
PR = B * D // 128
THR = 0.5


def _match_body(dbt_ref, plt_ref, g_ref, mm_ref, part_ref):
    dl = dbt_ref[0]
    dt = dbt_ref[1]
    dr = dbt_ref[2]
    db = dbt_ref[3]
    area_d = (dr - dl) * (db - dt)

    best = None
    bl = bt = br = bb = blab = None
    for j in range(NG):
        gl = g_ref[0, 0, 0, j]
        gtp = g_ref[0, 0, 1, j]
        gr = g_ref[0, 0, 2, j]
        gb = g_ref[0, 0, 3, j]
        lab = g_ref[0, 0, 4, j]
        area_g = (gr - gl) * (gb - gtp)
        iw = jnp.clip(jnp.minimum(dr, gr) - jnp.maximum(dl, gl), 0.0)
        ih = jnp.clip(jnp.minimum(db, gb) - jnp.maximum(dt, gtp), 0.0)
        inter = iw * ih
        iou = inter / (area_d + area_g - inter + 1e-8)
        if j == 0:
            best = iou
            bl = jnp.full_like(iou, gl)
            bt = jnp.full_like(iou, gtp)
            br = jnp.full_like(iou, gr)
            bb = jnp.full_like(iou, gb)
            blab = jnp.full_like(iou, lab)
        else:
            upd = iou > best
            best = jnp.where(upd, iou, best)
            bl = jnp.where(upd, gl, bl)
            bt = jnp.where(upd, gtp, bt)
            br = jnp.where(upd, gr, br)
            bb = jnp.where(upd, gb, bb)
            blab = jnp.where(upd, lab, blab)

    mask = best > THR
    maskf = mask.astype(jnp.float32)
    pos_b = jnp.sum(maskf)

    pl_ = plt_ref[0]
    sl1 = jnp.zeros_like(best)
    for c, bc in enumerate((bl, bt, br, bb)):
        dd = pl_[c] - bc
        adx = jnp.abs(dd)
        sl1 = sl1 + jnp.where(adx < 1.0, 0.5 * dd * dd, adx - 0.5)
    loss_l_b = jnp.sum(maskf * sl1)

    mm_ref[0, 0] = maskf
    mm_ref[0, 1] = jnp.where(mask, blab, 0.0)
    part_ref[0, 0, 0] = pos_b
    part_ref[0, 0, 1] = loss_l_b
    part_ref[0, 0, 2] = 0.0
    part_ref[0, 0, 3] = 0.0


def _ce_body(plabel_ref, mm_ref, y_ref):
    x = plabel_ref[0]
    mm = mm_ref[0]
    mlab_i = mm[:, 1:2].astype(jnp.int32)
    cls_lane = lax.broadcasted_iota(jnp.int32, (D, C), 1)
    ex = jnp.exp(x)
    xs = jnp.where(cls_lane == mlab_i, x, 0.0)
    ones = jnp.ones((C, 1), dtype=jnp.float32)
    se = lax.dot_general(ex, ones, (((1,), (0,)), ((), ())),
                         preferred_element_type=jnp.float32)
    pk = lax.dot_general(xs, ones, (((1,), (0,)), ((), ())),
                         preferred_element_type=jnp.float32)
    y_ref[0] = jnp.concatenate([se, pk], axis=1)


def _mining_body(se_ref, pk_ref, mf_ref, part_ref, out_ref, cn_ref, cnb_ref):
    se = se_ref[...]
    lse = jnp.log(se)
    loss_c = lse - pk_ref[...]
    mf = mf_ref[...]
    cn = (1.0 - mf) * loss_c
    pos_ce = jnp.sum(mf * loss_c)
    cn_ref[...] = cn
    cnb_ref[...] = lax.bitcast_convert_type(cn, jnp.int32)

    part = part_ref[...]
    pos = jnp.sum(part[:, :, 0:1])
    loss_l = jnp.sum(part[:, :, 1:2])
    k_i = (3.0 * pos).astype(jnp.int32)

    def search_step(i, t):
        cand = t | (jnp.int32(1) << (jnp.int32(30) - i))
        cnt = jnp.sum((cnb_ref[...] >= cand).astype(jnp.int32))
        return jnp.where(cnt >= k_i, cand, t)

    t_bits = lax.fori_loop(0, 31, search_step, jnp.int32(0))

    cnv = cn_ref[...]
    bits = cnb_ref[...]
    gt_m = bits > t_bits
    eq_m = bits == t_bits
    cnt_gt = jnp.sum(gt_m.astype(jnp.int32))
    s_gt = jnp.sum(jnp.where(gt_m, cnv, 0.0))
    cnt_eq = jnp.sum(eq_m.astype(jnp.int32))
    s_eq = jnp.sum(jnp.where(eq_m, cnv, 0.0))
    tval = s_eq / jnp.maximum(cnt_eq, 1).astype(jnp.float32)
    n_tie = jnp.clip(k_i - cnt_gt, 0, cnt_eq).astype(jnp.float32)
    neg_sum = s_gt + n_tie * tval
    neg_sum = jnp.where(k_i >= 1, neg_sum, 0.0)

    total = (loss_l + pos_ce + neg_sum) / jnp.maximum(pos, 1.0)
    out_ref[...] = jnp.full((1, 1), total, dtype=jnp.float32)


def kernel(ploc, plabel, gtloc, gtlabel, dboxes):
    f32 = jnp.float32
    dbt = jnp.pad(dboxes.T, ((0, 0), (0, DP - D))).reshape(4, R, 128)
    plt = jnp.pad(jnp.transpose(ploc, (0, 2, 1)),
                  ((0, 0), (0, 0), (0, DP - D))).reshape(B, 4, R, 128)
    g = jnp.concatenate(
        [jnp.transpose(gtloc, (0, 2, 1)),
         gtlabel.astype(f32)[:, None, :]], axis=1).reshape(B, 1, 5, NG)

    mm, part = pl.pallas_call(
        _match_body,
        grid=(B,),
        in_specs=[
            pl.BlockSpec((4, R, 128), lambda b: (0, 0, 0)),
            pl.BlockSpec((1, 4, R, 128), lambda b: (b, 0, 0, 0)),
            pl.BlockSpec((1, 1, 5, NG), lambda b: (b, 0, 0, 0),
                         memory_space=pltpu.SMEM),
        ],
        out_specs=[
            pl.BlockSpec((1, 2, R, 128), lambda b: (b, 0, 0, 0)),
            pl.BlockSpec((1, 1, 4), lambda b: (b, 0, 0),
                         memory_space=pltpu.SMEM),
        ],
        out_shape=[
            jax.ShapeDtypeStruct((B, 2, R, 128), f32),
            jax.ShapeDtypeStruct((B, 1, 4), f32),
        ],
        compiler_params=pltpu.CompilerParams(
            dimension_semantics=("arbitrary",),
        ),
    )(dbt, plt, g)

    mm_flat = mm.reshape(B, 2, DP)[:, :, :D]
    mm_dm = jnp.transpose(mm_flat, (0, 2, 1))

    y = pl.pallas_call(
        _ce_body,
        grid=(B,),
        in_specs=[
            pl.BlockSpec((1, D, C), lambda b: (b, 0, 0)),
            pl.BlockSpec((1, D, 2), lambda b: (b, 0, 0)),
        ],
        out_specs=pl.BlockSpec((1, D, 2), lambda b: (b, 0, 0)),
        out_shape=jax.ShapeDtypeStruct((B, D, 2), f32),
        compiler_params=pltpu.CompilerParams(
            dimension_semantics=("arbitrary",),
        ),
    )(plabel, mm_dm)

    se_p = y[:, :, 0].reshape(PR, 128)
    pk_p = y[:, :, 1].reshape(PR, 128)
    mf_p = mm_flat[:, 0, :].reshape(PR, 128)

    out = pl.pallas_call(
        _mining_body,
        out_shape=jax.ShapeDtypeStruct((1, 1), f32),
        scratch_shapes=[
            pltpu.VMEM((PR, 128), f32),
            pltpu.VMEM((PR, 128), jnp.int32),
        ],
    )(se_p, pk_p, mf_p, part)
    return out[0, 0]


# speedup vs baseline: 16.4286x; 1.2759x over previous
import jax
import jax.numpy as jnp
from jax import lax
from jax.experimental import pallas as pl
from jax.experimental.pallas import tpu as pltpu

B, D, NG, C = 64, 8732, 20, 81
DP = 8832
R = DP // 128
PR = B * D // 128
THR = 0.5


def _match_body(dbt_ref, plt_ref, g_ref, mk_ref, ml_ref, part_ref):
    dl = dbt_ref[0]
    dt = dbt_ref[1]
    dr = dbt_ref[2]
    db = dbt_ref[3]
    area_d = (dr - dl) * (db - dt)

    best = None
    bl = bt = br = bb = blab = None
    for j in range(NG):
        gl = g_ref[0, 0, 0, j]
        gtp = g_ref[0, 0, 1, j]
        gr = g_ref[0, 0, 2, j]
        gb = g_ref[0, 0, 3, j]
        lab = g_ref[0, 0, 4, j]
        area_g = (gr - gl) * (gb - gtp)
        iw = jnp.clip(jnp.minimum(dr, gr) - jnp.maximum(dl, gl), 0.0)
        ih = jnp.clip(jnp.minimum(db, gb) - jnp.maximum(dt, gtp), 0.0)
        inter = iw * ih
        iou = inter / (area_d + area_g - inter + 1e-8)
        if j == 0:
            best = iou
            bl = jnp.full_like(iou, gl)
            bt = jnp.full_like(iou, gtp)
            br = jnp.full_like(iou, gr)
            bb = jnp.full_like(iou, gb)
            blab = jnp.full_like(iou, lab)
        else:
            upd = iou > best
            best = jnp.where(upd, iou, best)
            bl = jnp.where(upd, gl, bl)
            bt = jnp.where(upd, gtp, bt)
            br = jnp.where(upd, gr, br)
            bb = jnp.where(upd, gb, bb)
            blab = jnp.where(upd, lab, blab)

    mask = best > THR
    maskf = mask.astype(jnp.float32)
    pos_b = jnp.sum(maskf)

    pl_ = plt_ref[0]
    sl1 = jnp.zeros_like(best)
    for c, bc in enumerate((bl, bt, br, bb)):
        dd = pl_[c] - bc
        adx = jnp.abs(dd)
        sl1 = sl1 + jnp.where(adx < 1.0, 0.5 * dd * dd, adx - 0.5)
    loss_l_b = jnp.sum(maskf * sl1)

    mk_ref[0] = maskf
    ml_ref[0] = jnp.where(mask, blab, 0.0)
    part_ref[0, 0, 0] = pos_b
    part_ref[0, 0, 1] = loss_l_b
    part_ref[0, 0, 2] = 0.0
    part_ref[0, 0, 3] = 0.0


def _ce_body(plabel_ref, ml_ref, se_ref, pk_ref):
    x = plabel_ref[0]
    mlab_i = ml_ref[0].astype(jnp.int32)
    cls_lane = lax.broadcasted_iota(jnp.int32, (D, C), 1)
    ex = jnp.exp(x)
    xs = jnp.where(cls_lane == mlab_i, x, 0.0)
    ones = jnp.ones((C, 1), dtype=jnp.float32)
    se_ref[0] = lax.dot_general(ex, ones, (((1,), (0,)), ((), ())),
                                preferred_element_type=jnp.float32)
    pk_ref[0] = lax.dot_general(xs, ones, (((1,), (0,)), ((), ())),
                                preferred_element_type=jnp.float32)


def _mining_body(se_ref, pk_ref, mf_ref, part_ref, out_ref, cn_ref, cnb_ref):
    se = se_ref[...]
    lse = jnp.log(se)
    loss_c = lse - pk_ref[...]
    mf = mf_ref[...]
    cn = (1.0 - mf) * loss_c
    pos_ce = jnp.sum(mf * loss_c)
    cn_ref[...] = cn
    cnb_ref[...] = lax.bitcast_convert_type(cn, jnp.int32)

    part = part_ref[...]
    pos = jnp.sum(part[:, :, 0:1])
    loss_l = jnp.sum(part[:, :, 1:2])
    k_i = (3.0 * pos).astype(jnp.int32)

    def search_step(i, t):
        cand = t | (jnp.int32(1) << (jnp.int32(30) - i))
        cnt = jnp.sum((cnb_ref[...] >= cand).astype(jnp.int32))
        return jnp.where(cnt >= k_i, cand, t)

    t_bits = lax.fori_loop(0, 31, search_step, jnp.int32(0))

    cnv = cn_ref[...]
    bits = cnb_ref[...]
    gt_m = bits > t_bits
    eq_m = bits == t_bits
    cnt_gt = jnp.sum(gt_m.astype(jnp.int32))
    s_gt = jnp.sum(jnp.where(gt_m, cnv, 0.0))
    cnt_eq = jnp.sum(eq_m.astype(jnp.int32))
    s_eq = jnp.sum(jnp.where(eq_m, cnv, 0.0))
    tval = s_eq / jnp.maximum(cnt_eq, 1).astype(jnp.float32)
    n_tie = jnp.clip(k_i - cnt_gt, 0, cnt_eq).astype(jnp.float32)
    neg_sum = s_gt + n_tie * tval
    neg_sum = jnp.where(k_i >= 1, neg_sum, 0.0)

    total = (loss_l + pos_ce + neg_sum) / jnp.maximum(pos, 1.0)
    out_ref[...] = jnp.full((1, 1), total, dtype=jnp.float32)


def kernel(ploc, plabel, gtloc, gtlabel, dboxes):
    f32 = jnp.float32
    dbt = jnp.pad(dboxes.T, ((0, 0), (0, DP - D))).reshape(4, R, 128)
    plt = jnp.pad(jnp.transpose(ploc, (0, 2, 1)),
                  ((0, 0), (0, 0), (0, DP - D))).reshape(B, 4, R, 128)
    g = jnp.concatenate(
        [jnp.transpose(gtloc, (0, 2, 1)),
         gtlabel.astype(f32)[:, None, :]], axis=1).reshape(B, 1, 5, NG)

    mk, ml, part = pl.pallas_call(
        _match_body,
        grid=(B,),
        in_specs=[
            pl.BlockSpec((4, R, 128), lambda b: (0, 0, 0)),
            pl.BlockSpec((1, 4, R, 128), lambda b: (b, 0, 0, 0)),
            pl.BlockSpec((1, 1, 5, NG), lambda b: (b, 0, 0, 0),
                         memory_space=pltpu.SMEM),
        ],
        out_specs=[
            pl.BlockSpec((1, R, 128), lambda b: (b, 0, 0)),
            pl.BlockSpec((1, R, 128), lambda b: (b, 0, 0)),
            pl.BlockSpec((1, 1, 4), lambda b: (b, 0, 0),
                         memory_space=pltpu.SMEM),
        ],
        out_shape=[
            jax.ShapeDtypeStruct((B, R, 128), f32),
            jax.ShapeDtypeStruct((B, R, 128), f32),
            jax.ShapeDtypeStruct((B, 1, 4), f32),
        ],
        compiler_params=pltpu.CompilerParams(
            dimension_semantics=("arbitrary",),
        ),
    )(dbt, plt, g)

    mk_dm = mk.reshape(B, DP)[:, :D, None]
    ml_dm = ml.reshape(B, DP)[:, :D, None]

    se, pk = pl.pallas_call(
        _ce_body,
        grid=(B,),
        in_specs=[
            pl.BlockSpec((1, D, C), lambda b: (b, 0, 0)),
            pl.BlockSpec((1, D, 1), lambda b: (b, 0, 0)),
        ],
        out_specs=[
            pl.BlockSpec((1, D, 1), lambda b: (b, 0, 0)),
            pl.BlockSpec((1, D, 1), lambda b: (b, 0, 0)),
        ],
        out_shape=[
            jax.ShapeDtypeStruct((B, D, 1), f32),
            jax.ShapeDtypeStruct((B, D, 1), f32),
        ],
        compiler_params=pltpu.CompilerParams(
            dimension_semantics=("arbitrary",),
        ),
    )(plabel, ml_dm)

    se_p = se.reshape(PR, 128)
    pk_p = pk.reshape(PR, 128)
    mf_p = mk_dm.reshape(PR, 128)

    out = pl.pallas_call(
        _mining_body,
        out_shape=jax.ShapeDtypeStruct((1, 1), f32),
        scratch_shapes=[
            pltpu.VMEM((PR, 128), f32),
            pltpu.VMEM((PR, 128), jnp.int32),
        ],
    )(se_p, pk_p, mf_p, part)
    return out[0, 0]
